# baseline (speedup 1.0000x reference)
import functools

import jax
import jax.numpy as jnp
from jax import lax
from jax.experimental import pallas as pl
from jax.experimental.pallas import tpu as pltpu
from jax.experimental.pallas import tpu_sc as plsc

_BATCH = 1024
_FEAT = 128
_CLASSES = 100000
_BETA = 0.05
_KNN = 6
_CB = 1024
_GRID = (_CLASSES + _CB - 1) // _CB
_NCHUNK = _CB // 128
_CHUNKS = _GRID * _NCHUNK
_NEG = -1e30


def _stats_body(x_ref, em_ref, t_ref, mm_ref, m_ref, s_ref):
    pid = pl.program_id(0)
    x = x_ref[...]
    em_blk = em_ref[...]

    t = lax.dot_general(
        x, em_blk, (((1,), (1,)), ((), ())),
        preferred_element_type=jnp.float32) / _BETA
    cols = pid * _CB + lax.broadcasted_iota(jnp.int32, (1, _CB), 1)
    t = jnp.where(cols < _CLASSES, t, _NEG)
    t_ref[...] = t.astype(jnp.bfloat16)

    @pl.when(pid == 0)
    def _():
        m_ref[...] = jnp.full((_BATCH, 1), _NEG, jnp.float32)
        s_ref[...] = jnp.zeros((_BATCH, 1), jnp.float32)

    bm = jnp.max(t, axis=1, keepdims=True)
    m_old = m_ref[...]
    m_new = jnp.maximum(m_old, bm)
    s_ref[...] = (s_ref[...] * jnp.exp(m_old - m_new)
                  + jnp.sum(jnp.exp(t - m_new), axis=1, keepdims=True))
    m_ref[...] = m_new

    cmax = [jnp.max(t[:, k * 128:(k + 1) * 128], axis=1, keepdims=True)
            for k in range(_NCHUNK)]
    mm_ref[...] = jnp.concatenate(cmax, axis=1).reshape(1, _BATCH, _NCHUNK)


def _select_body(mm_ref, idx_ref):
    work = mm_ref[...]
    iota = lax.broadcasted_iota(jnp.int32, (_BATCH, _CHUNKS), 1)
    picks = []
    for j in range(_KNN):
        mj = jnp.max(work, axis=1, keepdims=True)
        fi = jnp.min(jnp.where(work == mj, iota, jnp.int32(2**30)),
                     axis=1, keepdims=True)
        picks.append(fi)
        if j < _KNN - 1:
            work = jnp.where(iota == fi, _NEG, work)
    rowoff = lax.broadcasted_iota(jnp.int32, (_BATCH, 1), 0) * _CHUNKS
    idx_ref[...] = jnp.concatenate(picks, axis=1) + rowoff


def _loss_update_body(tc_ref, idx_ref, m_ref, s_ref, g_ref, x_ref, labc_ref,
                      labr_ref, a_ref, loss_ref, e_ref, il_ref):
    pairs = tc_ref[...].astype(jnp.float32)
    parity = jnp.bitwise_and(idx_ref[...], 1)
    halves = []
    for j in range(_KNN):
        pj = pairs[:, j * 256:(j + 1) * 256]
        odd = parity[:, j:j + 1] == 1
        halves.append(jnp.where(odd, pj[:, 128:], pj[:, :128]))
    work = jnp.concatenate(halves, axis=1)

    iota = lax.broadcasted_iota(jnp.int32, (_BATCH, _KNN * 128), 1)
    sum6 = jnp.zeros((_BATCH, 1), jnp.float32)
    mj = None
    for j in range(_KNN):
        mj = jnp.max(work, axis=1, keepdims=True)
        sum6 = sum6 + mj
        if j < _KNN - 1:
            fi = jnp.min(jnp.where(work == mj, iota, jnp.int32(2**30)),
                         axis=1, keepdims=True)
            work = jnp.where(iota == fi, _NEG, work)
    v6 = mj

    x = x_ref[...]
    g = g_ref[...]
    lse = m_ref[...] + jnp.log(s_ref[...])
    tl = jnp.sum(x * g, axis=1, keepdims=True) / _BETA
    it = (tl >= v6).astype(jnp.float32)
    loss_i = (8.0 - it) * lse - sum6 - (2.0 - it) * tl
    loss_ref[...] = jnp.sum(loss_i, axis=0, keepdims=True) / _BATCH

    labc = labc_ref[...]
    labr = labr_ref[...]
    ii = lax.broadcasted_iota(jnp.int32, (_BATCH, _BATCH), 0)
    jj = lax.broadcasted_iota(jnp.int32, (_BATCH, _BATCH), 1)
    eq = labc == labr
    pm = jnp.logical_and(eq, jj < ii)
    haspred = jnp.any(pm, axis=1, keepdims=True)
    p = jnp.max(jnp.where(pm, jj, -1), axis=1, keepdims=True)
    occ = jnp.sum(jnp.where(pm, 1, 0), axis=1, keepdims=True)
    maxocc = jnp.max(occ)

    lasti = jnp.max(jnp.where(eq, ii, -1), axis=0, keepdims=True)
    il_ref[...] = (lasti == lax.broadcasted_iota(
        jnp.int32, (1, _BATCH), 1)).astype(jnp.float32)

    alpha = a_ref[0, 0]

    def _norm(v):
        return v / jnp.sqrt(jnp.sum(v * v, axis=1, keepdims=True))

    e_ref[...] = _norm(alpha * g + (1.0 - alpha) * x)
    onehot = (jj == p).astype(jnp.float32)

    def body(_, carry):
        e = e_ref[...]
        eprev = lax.dot_general(
            onehot, e, (((1,), (0,)), ((), ())),
            preferred_element_type=jnp.float32)
        base = jnp.where(haspred, eprev, g)
        e_ref[...] = _norm(alpha * base + (1.0 - alpha) * x)
        return carry

    lax.fori_loop(0, maxocc, body, 0)


def _scatter_body(em_ref, e_ref, labr_ref, il_ref, out_ref):
    pid = pl.program_id(0)
    c = pid * _CB + lax.broadcasted_iota(jnp.int32, (_CB, 1), 0)
    sel = jnp.logical_and(labr_ref[...] == c, il_ref[...] > 0.0)
    self32 = sel.astype(jnp.float32)
    val = lax.dot_general(
        self32, e_ref[...], (((1,), (0,)), ((), ())),
        preferred_element_type=jnp.float32)
    hit = jnp.max(self32, axis=1, keepdims=True)
    out_ref[...] = jnp.where(hit > 0.0, val, em_ref[...])


def _make_sc_gather(V, D, B):
    num_cores, num_subcores = 2, 16
    nw = num_cores * num_subcores
    b_per_w = B // nw
    mesh = plsc.VectorSubcoreMesh(
        core_axis_name="c", subcore_axis_name="s",
        num_cores=num_cores, num_subcores=num_subcores)

    @functools.partial(
        pl.kernel, mesh=mesh,
        out_type=jax.ShapeDtypeStruct((B, D), jnp.float32),
        scratch_types=[
            pltpu.VMEM((b_per_w,), jnp.int32),
            pltpu.VMEM((b_per_w, D), jnp.float32),
            pltpu.SemaphoreType.DMA,
        ],
    )
    def k(table_hbm, idx_hbm, out_hbm, idx_v, rows_v, sem):
        wid = lax.axis_index("s") * num_cores + lax.axis_index("c")
        base = wid * b_per_w
        pltpu.sync_copy(idx_hbm.at[pl.ds(base, b_per_w)], idx_v)
        pltpu.async_copy(table_hbm.at[idx_v], rows_v, sem).wait()
        pltpu.sync_copy(rows_v, out_hbm.at[pl.ds(base, b_per_w)])

    return k


_sc_gather_cache = {}


def _sc_gather(table, idx):
    key = (table.shape, idx.shape)
    if key not in _sc_gather_cache:
        _sc_gather_cache[key] = _make_sc_gather(
            table.shape[0], table.shape[1], idx.shape[0])
    return _sc_gather_cache[key](table, idx)


def kernel(inputs, label, epoch, em):
    x = inputs.astype(jnp.float32)
    labc = label.reshape(_BATCH, 1)
    labr = label.reshape(1, _BATCH)
    alpha = (0.01 * epoch) * jnp.ones((1, 1), jnp.float32)

    t_store, mm, m, s = pl.pallas_call(
        _stats_body,
        grid=(_GRID,),
        in_specs=[
            pl.BlockSpec((_BATCH, _FEAT), lambda i: (0, 0)),
            pl.BlockSpec((_CB, _FEAT), lambda i: (i, 0)),
        ],
        out_specs=[
            pl.BlockSpec((_BATCH, _CB), lambda i: (0, i)),
            pl.BlockSpec((1, _BATCH, _NCHUNK), lambda i: (i, 0, 0)),
            pl.BlockSpec((_BATCH, 1), lambda i: (0, 0)),
            pl.BlockSpec((_BATCH, 1), lambda i: (0, 0)),
        ],
        out_shape=[
            jax.ShapeDtypeStruct((_BATCH, _GRID * _CB), jnp.bfloat16),
            jax.ShapeDtypeStruct((_GRID, _BATCH, _NCHUNK), jnp.float32),
            jax.ShapeDtypeStruct((_BATCH, 1), jnp.float32),
            jax.ShapeDtypeStruct((_BATCH, 1), jnp.float32),
        ],
    )(x, em)

    g = _sc_gather(em, label)

    mm2 = mm.transpose(1, 0, 2).reshape(_BATCH, _CHUNKS)
    idx = pl.pallas_call(
        _select_body,
        in_specs=[pl.BlockSpec((_BATCH, _CHUNKS), lambda: (0, 0))],
        out_specs=pl.BlockSpec((_BATCH, _KNN), lambda: (0, 0)),
        out_shape=jax.ShapeDtypeStruct((_BATCH, _KNN), jnp.int32),
    )(mm2)

    t_rows = jax.lax.bitcast_convert_type(
        t_store.reshape(_BATCH * _CHUNKS // 2, 128, 2),
        jnp.float32)
    idx_half = idx.reshape(_BATCH * _KNN) // 2
    tc32 = _sc_gather(t_rows, idx_half)
    tc = jax.lax.bitcast_convert_type(tc32, jnp.bfloat16).reshape(
        _BATCH, _KNN * 256)

    loss, e_rows, il = pl.pallas_call(
        _loss_update_body,
        in_specs=[
            pl.BlockSpec((_BATCH, _KNN * 256), lambda: (0, 0)),
            pl.BlockSpec((_BATCH, _KNN), lambda: (0, 0)),
            pl.BlockSpec((_BATCH, 1), lambda: (0, 0)),
            pl.BlockSpec((_BATCH, 1), lambda: (0, 0)),
            pl.BlockSpec((_BATCH, _FEAT), lambda: (0, 0)),
            pl.BlockSpec((_BATCH, _FEAT), lambda: (0, 0)),
            pl.BlockSpec((_BATCH, 1), lambda: (0, 0)),
            pl.BlockSpec((1, _BATCH), lambda: (0, 0)),
            pl.BlockSpec(memory_space=pltpu.SMEM),
        ],
        out_specs=[
            pl.BlockSpec((1, 1), lambda: (0, 0)),
            pl.BlockSpec((_BATCH, _FEAT), lambda: (0, 0)),
            pl.BlockSpec((1, _BATCH), lambda: (0, 0)),
        ],
        out_shape=[
            jax.ShapeDtypeStruct((1, 1), jnp.float32),
            jax.ShapeDtypeStruct((_BATCH, _FEAT), jnp.float32),
            jax.ShapeDtypeStruct((1, _BATCH), jnp.float32),
        ],
    )(tc, idx, m, s, g, x, labc, labr, alpha)

    new_em = pl.pallas_call(
        _scatter_body,
        grid=(_GRID,),
        in_specs=[
            pl.BlockSpec((_CB, _FEAT), lambda i: (i, 0)),
            pl.BlockSpec((_BATCH, _FEAT), lambda i: (0, 0)),
            pl.BlockSpec((1, _BATCH), lambda i: (0, 0)),
            pl.BlockSpec((1, _BATCH), lambda i: (0, 0)),
        ],
        out_specs=pl.BlockSpec((_CB, _FEAT), lambda i: (i, 0)),
        out_shape=jax.ShapeDtypeStruct((_CLASSES, _FEAT), jnp.float32),
    )(em, e_rows, labr, il)

    return loss[0, 0], new_em


# speedup vs baseline: 98.6632x; 98.6632x over previous
import functools

import jax
import jax.numpy as jnp
from jax import lax
from jax.experimental import pallas as pl
from jax.experimental.pallas import tpu as pltpu
from jax.experimental.pallas import tpu_sc as plsc

_BATCH = 1024
_FEAT = 128
_CLASSES = 100000
_BETA = 0.05
_KNN = 6
_CB = 1024
_GRID = (_CLASSES + _CB - 1) // _CB
_NCHUNK = _CB // 128
_CHUNKS = _GRID * _NCHUNK
_NEG = -1e30


def _stats_body(x_ref, em_ref, t_ref, mm_ref, m_ref, s_ref):
    pid = pl.program_id(0)
    x = x_ref[...]
    em_blk = em_ref[...]

    t = lax.dot_general(
        x, em_blk, (((1,), (1,)), ((), ())),
        preferred_element_type=jnp.float32) / _BETA
    cols = pid * _CB + lax.broadcasted_iota(jnp.int32, (1, _CB), 1)
    t = jnp.where(cols < _CLASSES, t, _NEG)
    for k in range(_NCHUNK):
        t_ref[k * _BATCH:(k + 1) * _BATCH, :] = t[:, k * 128:(k + 1) * 128]

    @pl.when(pid == 0)
    def _():
        m_ref[...] = jnp.full((_BATCH, 1), _NEG, jnp.float32)
        s_ref[...] = jnp.zeros((_BATCH, 1), jnp.float32)

    bm = jnp.max(t, axis=1, keepdims=True)
    m_old = m_ref[...]
    m_new = jnp.maximum(m_old, bm)
    s_ref[...] = (s_ref[...] * jnp.exp(m_old - m_new)
                  + jnp.sum(jnp.exp(t - m_new), axis=1, keepdims=True))
    m_ref[...] = m_new

    cmax = [jnp.max(t[:, k * 128:(k + 1) * 128], axis=1, keepdims=True)
            for k in range(_NCHUNK)]
    mm_ref[...] = jnp.concatenate(cmax, axis=1).reshape(1, _BATCH, _NCHUNK)


def _select_body(mm_ref, idx_ref):
    work = mm_ref[...]
    iota = lax.broadcasted_iota(jnp.int32, (_BATCH, _CHUNKS), 1)
    picks = []
    for j in range(_KNN):
        mj = jnp.max(work, axis=1, keepdims=True)
        fi = jnp.min(jnp.where(work == mj, iota, jnp.int32(2**30)),
                     axis=1, keepdims=True)
        picks.append(fi)
        if j < _KNN - 1:
            work = jnp.where(iota == fi, _NEG, work)
    rowid = lax.broadcasted_iota(jnp.int32, (_BATCH, 1), 0)
    idx_ref[...] = jnp.concatenate(picks, axis=1) * _BATCH + rowid


def _loss_update_body(tc_ref, m_ref, s_ref, g_ref, x_ref, labc_ref,
                      labr_ref, a_ref, loss_ref, e_ref, il_ref):
    work = tc_ref[...]
    iota = lax.broadcasted_iota(jnp.int32, (_BATCH, _KNN * 128), 1)
    sum6 = jnp.zeros((_BATCH, 1), jnp.float32)
    mj = None
    for j in range(_KNN):
        mj = jnp.max(work, axis=1, keepdims=True)
        sum6 = sum6 + mj
        if j < _KNN - 1:
            fi = jnp.min(jnp.where(work == mj, iota, jnp.int32(2**30)),
                         axis=1, keepdims=True)
            work = jnp.where(iota == fi, _NEG, work)
    v6 = mj

    x = x_ref[...]
    g = g_ref[...]
    lse = m_ref[...] + jnp.log(s_ref[...])
    tl = jnp.sum(x * g, axis=1, keepdims=True) / _BETA
    it = (tl >= v6).astype(jnp.float32)
    loss_i = (8.0 - it) * lse - sum6 - (2.0 - it) * tl
    loss_ref[...] = jnp.sum(loss_i, axis=0, keepdims=True) / _BATCH

    labc = labc_ref[...]
    labr = labr_ref[...]
    ii = lax.broadcasted_iota(jnp.int32, (_BATCH, _BATCH), 0)
    jj = lax.broadcasted_iota(jnp.int32, (_BATCH, _BATCH), 1)
    eq = labc == labr
    pm = jnp.logical_and(eq, jj < ii)
    haspred = jnp.any(pm, axis=1, keepdims=True)
    p = jnp.max(jnp.where(pm, jj, -1), axis=1, keepdims=True)
    occ = jnp.sum(jnp.where(pm, 1, 0), axis=1, keepdims=True)
    maxocc = jnp.max(occ)

    lasti = jnp.max(jnp.where(eq, ii, -1), axis=0, keepdims=True)
    il_ref[...] = (lasti == lax.broadcasted_iota(
        jnp.int32, (1, _BATCH), 1)).astype(jnp.float32)

    alpha = a_ref[0, 0]

    def _norm(v):
        return v / jnp.sqrt(jnp.sum(v * v, axis=1, keepdims=True))

    e_ref[...] = _norm(alpha * g + (1.0 - alpha) * x)
    onehot = (jj == p).astype(jnp.float32)

    def body(_, carry):
        e = e_ref[...]
        eprev = lax.dot_general(
            onehot, e, (((1,), (0,)), ((), ())),
            preferred_element_type=jnp.float32)
        base = jnp.where(haspred, eprev, g)
        e_ref[...] = _norm(alpha * base + (1.0 - alpha) * x)
        return carry

    lax.fori_loop(0, maxocc, body, 0)


def _scatter_body(em_ref, e_ref, labr_ref, il_ref, out_ref):
    pid = pl.program_id(0)
    c = pid * _CB + lax.broadcasted_iota(jnp.int32, (_CB, 1), 0)
    sel = jnp.logical_and(labr_ref[...] == c, il_ref[...] > 0.0)
    self32 = sel.astype(jnp.float32)
    val = lax.dot_general(
        self32, e_ref[...], (((1,), (0,)), ((), ())),
        preferred_element_type=jnp.float32)
    hit = jnp.max(self32, axis=1, keepdims=True)
    out_ref[...] = jnp.where(hit > 0.0, val, em_ref[...])


def _make_sc_gather(V, D, B):
    num_cores, num_subcores = 2, 16
    nw = num_cores * num_subcores
    b_per_w = B // nw
    mesh = plsc.VectorSubcoreMesh(
        core_axis_name="c", subcore_axis_name="s",
        num_cores=num_cores, num_subcores=num_subcores)

    @functools.partial(
        pl.kernel, mesh=mesh,
        out_type=jax.ShapeDtypeStruct((B, D), jnp.float32),
        scratch_types=[
            pltpu.VMEM((b_per_w,), jnp.int32),
            pltpu.VMEM((b_per_w, D), jnp.float32),
            pltpu.SemaphoreType.DMA,
        ],
    )
    def k(table_hbm, idx_hbm, out_hbm, idx_v, rows_v, sem):
        wid = lax.axis_index("s") * num_cores + lax.axis_index("c")
        base = wid * b_per_w
        pltpu.sync_copy(idx_hbm.at[pl.ds(base, b_per_w)], idx_v)
        pltpu.async_copy(table_hbm.at[idx_v], rows_v, sem).wait()
        pltpu.sync_copy(rows_v, out_hbm.at[pl.ds(base, b_per_w)])

    return k


_sc_gather_cache = {}


def _sc_gather(table, idx):
    key = (table.shape, idx.shape)
    if key not in _sc_gather_cache:
        _sc_gather_cache[key] = _make_sc_gather(
            table.shape[0], table.shape[1], idx.shape[0])
    return _sc_gather_cache[key](table, idx)


def kernel(inputs, label, epoch, em):
    x = inputs.astype(jnp.float32)
    labc = label.reshape(_BATCH, 1)
    labr = label.reshape(1, _BATCH)
    alpha = (0.01 * epoch) * jnp.ones((1, 1), jnp.float32)

    t_store, mm, m, s = pl.pallas_call(
        _stats_body,
        grid=(_GRID,),
        in_specs=[
            pl.BlockSpec((_BATCH, _FEAT), lambda i: (0, 0)),
            pl.BlockSpec((_CB, _FEAT), lambda i: (i, 0)),
        ],
        out_specs=[
            pl.BlockSpec((_NCHUNK * _BATCH, 128), lambda i: (i, 0)),
            pl.BlockSpec((1, _BATCH, _NCHUNK), lambda i: (i, 0, 0)),
            pl.BlockSpec((_BATCH, 1), lambda i: (0, 0)),
            pl.BlockSpec((_BATCH, 1), lambda i: (0, 0)),
        ],
        out_shape=[
            jax.ShapeDtypeStruct((_CHUNKS * _BATCH, 128), jnp.float32),
            jax.ShapeDtypeStruct((_GRID, _BATCH, _NCHUNK), jnp.float32),
            jax.ShapeDtypeStruct((_BATCH, 1), jnp.float32),
            jax.ShapeDtypeStruct((_BATCH, 1), jnp.float32),
        ],
    )(x, em)

    g = _sc_gather(em, label)

    mm2 = mm.transpose(1, 0, 2).reshape(_BATCH, _CHUNKS)
    idx = pl.pallas_call(
        _select_body,
        in_specs=[pl.BlockSpec((_BATCH, _CHUNKS), lambda: (0, 0))],
        out_specs=pl.BlockSpec((_BATCH, _KNN), lambda: (0, 0)),
        out_shape=jax.ShapeDtypeStruct((_BATCH, _KNN), jnp.int32),
    )(mm2)

    tc = _sc_gather(t_store, idx.reshape(_BATCH * _KNN))
    tc = tc.reshape(_BATCH, _KNN * 128)

    loss, e_rows, il = pl.pallas_call(
        _loss_update_body,
        in_specs=[
            pl.BlockSpec((_BATCH, _KNN * 128), lambda: (0, 0)),
            pl.BlockSpec((_BATCH, 1), lambda: (0, 0)),
            pl.BlockSpec((_BATCH, 1), lambda: (0, 0)),
            pl.BlockSpec((_BATCH, _FEAT), lambda: (0, 0)),
            pl.BlockSpec((_BATCH, _FEAT), lambda: (0, 0)),
            pl.BlockSpec((_BATCH, 1), lambda: (0, 0)),
            pl.BlockSpec((1, _BATCH), lambda: (0, 0)),
            pl.BlockSpec(memory_space=pltpu.SMEM),
        ],
        out_specs=[
            pl.BlockSpec((1, 1), lambda: (0, 0)),
            pl.BlockSpec((_BATCH, _FEAT), lambda: (0, 0)),
            pl.BlockSpec((1, _BATCH), lambda: (0, 0)),
        ],
        out_shape=[
            jax.ShapeDtypeStruct((1, 1), jnp.float32),
            jax.ShapeDtypeStruct((_BATCH, _FEAT), jnp.float32),
            jax.ShapeDtypeStruct((1, _BATCH), jnp.float32),
        ],
    )(tc, m, s, g, x, labc, labr, alpha)

    new_em = pl.pallas_call(
        _scatter_body,
        grid=(_GRID,),
        in_specs=[
            pl.BlockSpec((_CB, _FEAT), lambda i: (i, 0)),
            pl.BlockSpec((_BATCH, _FEAT), lambda i: (0, 0)),
            pl.BlockSpec((1, _BATCH), lambda i: (0, 0)),
            pl.BlockSpec((1, _BATCH), lambda i: (0, 0)),
        ],
        out_specs=pl.BlockSpec((_CB, _FEAT), lambda i: (i, 0)),
        out_shape=jax.ShapeDtypeStruct((_CLASSES, _FEAT), jnp.float32),
    )(em, e_rows, labr, il)

    return loss[0, 0], new_em


# speedup vs baseline: 129.7720x; 1.3153x over previous
import functools

import jax
import jax.numpy as jnp
from jax import lax
from jax.experimental import pallas as pl
from jax.experimental.pallas import tpu as pltpu
from jax.experimental.pallas import tpu_sc as plsc

_BATCH = 1024
_FEAT = 128
_CLASSES = 100000
_BETA = 0.05
_KNN = 6
_CB = 1024
_GRID = (_CLASSES + _CB - 1) // _CB
_NCHUNK = _CB // 128
_CHUNKS = _GRID * _NCHUNK
_NEG = -1e30


def _stats_body(x_ref, em_ref, t_ref, mm_ref, m_ref, s_ref, copy_ref):
    pid = pl.program_id(0)
    x = x_ref[...]
    em_blk = em_ref[...]
    copy_ref[...] = em_blk

    t = lax.dot_general(
        x, em_blk, (((1,), (1,)), ((), ())),
        preferred_element_type=jnp.float32) / _BETA
    cols = pid * _CB + lax.broadcasted_iota(jnp.int32, (1, _CB), 1)
    t = jnp.where(cols < _CLASSES, t, _NEG)
    for k in range(_NCHUNK):
        t_ref[k * _BATCH:(k + 1) * _BATCH, :] = t[:, k * 128:(k + 1) * 128]

    @pl.when(pid == 0)
    def _():
        m_ref[...] = jnp.full((_BATCH, 1), _NEG, jnp.float32)
        s_ref[...] = jnp.zeros((_BATCH, 1), jnp.float32)

    bm = jnp.max(t, axis=1, keepdims=True)
    m_old = m_ref[...]
    m_new = jnp.maximum(m_old, bm)
    s_ref[...] = (s_ref[...] * jnp.exp(m_old - m_new)
                  + jnp.sum(jnp.exp(t - m_new), axis=1, keepdims=True))
    m_ref[...] = m_new

    cmax = [jnp.max(t[:, k * 128:(k + 1) * 128], axis=1, keepdims=True)
            for k in range(_NCHUNK)]
    mm_ref[...] = jnp.concatenate(cmax, axis=1).reshape(1, _BATCH, _NCHUNK)


def _select_body(mm_ref, idx_ref):
    work = mm_ref[...]
    iota = lax.broadcasted_iota(jnp.int32, (_BATCH, _CHUNKS), 1)
    picks = []
    for j in range(_KNN):
        mj = jnp.max(work, axis=1, keepdims=True)
        fi = jnp.min(jnp.where(work == mj, iota, jnp.int32(2**30)),
                     axis=1, keepdims=True)
        picks.append(fi)
        if j < _KNN - 1:
            work = jnp.where(iota == fi, _NEG, work)
    rowid = lax.broadcasted_iota(jnp.int32, (_BATCH, 1), 0)
    idx_ref[...] = jnp.concatenate(picks, axis=1) * _BATCH + rowid


def _loss_update_body(tc_ref, m_ref, s_ref, g_ref, x_ref, labc_ref,
                      labr_ref, a_ref, loss_ref, e_ref, es_ref):
    work = tc_ref[...]
    iota = lax.broadcasted_iota(jnp.int32, (_BATCH, _KNN * 128), 1)
    sum6 = jnp.zeros((_BATCH, 1), jnp.float32)
    mj = None
    for j in range(_KNN):
        mj = jnp.max(work, axis=1, keepdims=True)
        sum6 = sum6 + mj
        if j < _KNN - 1:
            fi = jnp.min(jnp.where(work == mj, iota, jnp.int32(2**30)),
                         axis=1, keepdims=True)
            work = jnp.where(iota == fi, _NEG, work)
    v6 = mj

    x = x_ref[...]
    g = g_ref[...]
    lse = m_ref[...] + jnp.log(s_ref[...])
    tl = jnp.sum(x * g, axis=1, keepdims=True) / _BETA
    it = (tl >= v6).astype(jnp.float32)
    loss_i = (8.0 - it) * lse - sum6 - (2.0 - it) * tl
    loss_ref[...] = jnp.sum(loss_i, axis=0, keepdims=True) / _BATCH

    labc = labc_ref[...]
    labr = labr_ref[...]
    ii = lax.broadcasted_iota(jnp.int32, (_BATCH, _BATCH), 0)
    jj = lax.broadcasted_iota(jnp.int32, (_BATCH, _BATCH), 1)
    eq = labc == labr
    pm = jnp.logical_and(eq, jj < ii)
    haspred = jnp.any(pm, axis=1, keepdims=True)
    p = jnp.max(jnp.where(pm, jj, -1), axis=1, keepdims=True)
    occ = jnp.sum(jnp.where(pm, 1, 0), axis=1, keepdims=True)
    maxocc = jnp.max(occ)
    lastj = jnp.max(jnp.where(eq, jj, -1), axis=1, keepdims=True)

    alpha = a_ref[0, 0]

    def _norm(v):
        return v / jnp.sqrt(jnp.sum(v * v, axis=1, keepdims=True))

    e_ref[...] = _norm(alpha * g + (1.0 - alpha) * x)
    onehot = (jj == p).astype(jnp.float32)

    def body(_, carry):
        e = e_ref[...]
        eprev = lax.dot_general(
            onehot, e, (((1,), (0,)), ((), ())),
            preferred_element_type=jnp.float32)
        base = jnp.where(haspred, eprev, g)
        e_ref[...] = _norm(alpha * base + (1.0 - alpha) * x)
        return carry

    lax.fori_loop(0, maxocc, body, 0)

    onehot_last = (jj == lastj).astype(jnp.float32)
    es_ref[...] = lax.dot_general(
        onehot_last, e_ref[...], (((1,), (0,)), ((), ())),
        preferred_element_type=jnp.float32)


def _make_sc_scatter(V, D, B):
    num_cores, num_subcores = 2, 16
    nw = num_cores * num_subcores
    b_per_w = B // nw
    mesh = plsc.VectorSubcoreMesh(
        core_axis_name="c", subcore_axis_name="s",
        num_cores=num_cores, num_subcores=num_subcores)

    @functools.partial(
        pl.kernel, mesh=mesh,
        out_type=(),
        scratch_types=[
            pltpu.VMEM((b_per_w,), jnp.int32),
            pltpu.VMEM((b_per_w, D), jnp.float32),
            pltpu.SemaphoreType.DMA,
        ],
    )
    def k(rows_hbm, idx_hbm, table_ref, idx_v, rows_v, sem):
        wid = lax.axis_index("s") * num_cores + lax.axis_index("c")
        base = wid * b_per_w
        pltpu.sync_copy(idx_hbm.at[pl.ds(base, b_per_w)], idx_v)
        pltpu.sync_copy(rows_hbm.at[pl.ds(base, b_per_w)], rows_v)
        pltpu.async_copy(rows_v, table_ref.at[idx_v], sem).wait()

    return k


_sc_scatter_fn = None


def _sc_scatter(rows, idx, table_ref):
    global _sc_scatter_fn
    if _sc_scatter_fn is None:
        _sc_scatter_fn = _make_sc_scatter(
            _CLASSES, _FEAT, _BATCH)
    _sc_scatter_fn(rows, idx, table_ref)


def _make_sc_gather(V, D, B):
    num_cores, num_subcores = 2, 16
    nw = num_cores * num_subcores
    b_per_w = B // nw
    mesh = plsc.VectorSubcoreMesh(
        core_axis_name="c", subcore_axis_name="s",
        num_cores=num_cores, num_subcores=num_subcores)

    @functools.partial(
        pl.kernel, mesh=mesh,
        out_type=jax.ShapeDtypeStruct((B, D), jnp.float32),
        scratch_types=[
            pltpu.VMEM((b_per_w,), jnp.int32),
            pltpu.VMEM((b_per_w, D), jnp.float32),
            pltpu.SemaphoreType.DMA,
        ],
    )
    def k(table_hbm, idx_hbm, out_hbm, idx_v, rows_v, sem):
        wid = lax.axis_index("s") * num_cores + lax.axis_index("c")
        base = wid * b_per_w
        pltpu.sync_copy(idx_hbm.at[pl.ds(base, b_per_w)], idx_v)
        pltpu.async_copy(table_hbm.at[idx_v], rows_v, sem).wait()
        pltpu.sync_copy(rows_v, out_hbm.at[pl.ds(base, b_per_w)])

    return k


_sc_gather_cache = {}


def _sc_gather(table, idx):
    key = (table.shape, idx.shape)
    if key not in _sc_gather_cache:
        _sc_gather_cache[key] = _make_sc_gather(
            table.shape[0], table.shape[1], idx.shape[0])
    return _sc_gather_cache[key](table, idx)


def kernel(inputs, label, epoch, em):
    x = inputs.astype(jnp.float32)
    labc = label.reshape(_BATCH, 1)
    labr = label.reshape(1, _BATCH)
    alpha = (0.01 * epoch) * jnp.ones((1, 1), jnp.float32)

    t_store, mm, m, s, base = pl.pallas_call(
        _stats_body,
        grid=(_GRID,),
        in_specs=[
            pl.BlockSpec((_BATCH, _FEAT), lambda i: (0, 0)),
            pl.BlockSpec((_CB, _FEAT), lambda i: (i, 0)),
        ],
        out_specs=[
            pl.BlockSpec((_NCHUNK * _BATCH, 128), lambda i: (i, 0)),
            pl.BlockSpec((1, _BATCH, _NCHUNK), lambda i: (i, 0, 0)),
            pl.BlockSpec((_BATCH, 1), lambda i: (0, 0)),
            pl.BlockSpec((_BATCH, 1), lambda i: (0, 0)),
            pl.BlockSpec((_CB, _FEAT), lambda i: (i, 0)),
        ],
        out_shape=[
            jax.ShapeDtypeStruct((_CHUNKS * _BATCH, 128), jnp.float32),
            jax.ShapeDtypeStruct((_GRID, _BATCH, _NCHUNK), jnp.float32),
            jax.ShapeDtypeStruct((_BATCH, 1), jnp.float32),
            jax.ShapeDtypeStruct((_BATCH, 1), jnp.float32),
            jax.ShapeDtypeStruct((_CLASSES, _FEAT), jnp.float32),
        ],
    )(x, em)

    g = _sc_gather(em, label)

    mm2 = mm.transpose(1, 0, 2).reshape(_BATCH, _CHUNKS)
    idx = pl.pallas_call(
        _select_body,
        in_specs=[pl.BlockSpec((_BATCH, _CHUNKS), lambda: (0, 0))],
        out_specs=pl.BlockSpec((_BATCH, _KNN), lambda: (0, 0)),
        out_shape=jax.ShapeDtypeStruct((_BATCH, _KNN), jnp.int32),
    )(mm2)

    tc = _sc_gather(t_store, idx.reshape(_BATCH * _KNN))
    tc = tc.reshape(_BATCH, _KNN * 128)

    loss, e_rows, e_scat = pl.pallas_call(
        _loss_update_body,
        in_specs=[
            pl.BlockSpec((_BATCH, _KNN * 128), lambda: (0, 0)),
            pl.BlockSpec((_BATCH, 1), lambda: (0, 0)),
            pl.BlockSpec((_BATCH, 1), lambda: (0, 0)),
            pl.BlockSpec((_BATCH, _FEAT), lambda: (0, 0)),
            pl.BlockSpec((_BATCH, _FEAT), lambda: (0, 0)),
            pl.BlockSpec((_BATCH, 1), lambda: (0, 0)),
            pl.BlockSpec((1, _BATCH), lambda: (0, 0)),
            pl.BlockSpec(memory_space=pltpu.SMEM),
        ],
        out_specs=[
            pl.BlockSpec((1, 1), lambda: (0, 0)),
            pl.BlockSpec((_BATCH, _FEAT), lambda: (0, 0)),
            pl.BlockSpec((_BATCH, _FEAT), lambda: (0, 0)),
        ],
        out_shape=[
            jax.ShapeDtypeStruct((1, 1), jnp.float32),
            jax.ShapeDtypeStruct((_BATCH, _FEAT), jnp.float32),
            jax.ShapeDtypeStruct((_BATCH, _FEAT), jnp.float32),
        ],
    )(tc, m, s, g, x, labc, labr, alpha)

    base_ref = jax.new_ref(base)
    _sc_scatter(e_scat, label, base_ref)
    new_em = base_ref[...]

    return loss[0, 0], new_em
